# baseline (speedup 1.0000x reference)
import functools

import jax
import jax.numpy as jnp
from jax import lax
from jax.experimental import pallas as pl
from jax.experimental.pallas import tpu as pltpu
from jax.experimental.pallas import tpu_sc as plsc

EMB_DIM = 64
LABELS = 64

_NC = 2
_NS = 16
_NW = _NC * _NS
_CHUNK = 128
_SUB = 8


@functools.lru_cache(maxsize=None)
def _make_gather(B):
    b_per_w = B // _NW
    n_pass = b_per_w // _CHUNK
    mesh = plsc.VectorSubcoreMesh(core_axis_name="c", subcore_axis_name="s")

    @functools.partial(
        pl.kernel,
        mesh=mesh,
        out_type=[
            jax.ShapeDtypeStruct((B, EMB_DIM), jnp.float32),
            jax.ShapeDtypeStruct((B, EMB_DIM), jnp.float32),
        ],
        scratch_types=[
            pltpu.VMEM((b_per_w,), jnp.int32),
            pltpu.VMEM((b_per_w,), jnp.int32),
            pltpu.VMEM((b_per_w // 2, EMB_DIM), jnp.float32),
            pltpu.SemaphoreType.DMA,
            pltpu.SemaphoreType.DMA,
        ],
    )
    def gather(cells_hbm, genes_hbm, cell_tab, gene_tab, cell_out, gene_out,
               idx_c, idx_g, stage, sem_s, sem_d):
        wid = lax.axis_index("s") * _NC + lax.axis_index("c")
        base = wid * b_per_w
        half = b_per_w // 2
        pltpu.sync_copy(cells_hbm.at[pl.ds(base, b_per_w)], idx_c)
        pltpu.sync_copy(genes_hbm.at[pl.ds(base, b_per_w)], idx_g)

        for tab, idx, out in (
            (cell_tab, idx_c, cell_out),
            (gene_tab, idx_g, gene_out),
        ):
            def body(g, carry):
                j0 = g * 16
                vec_s = idx[pl.ds(j0, 16)]
                vec_d = idx[pl.ds(half + j0, 16)]
                for k in range(16):
                    pltpu.make_async_copy(
                        tab.at[pl.ds(vec_s[k], 1)],
                        stage.at[pl.ds(j0 + k, 1)], sem_s).start()
                    pltpu.make_async_copy(
                        tab.at[pl.ds(vec_d[k], 1)],
                        out.at[pl.ds(base + half + j0 + k, 1)], sem_d).start()
                return carry

            lax.fori_loop(0, half // 16, body, 0)
            pltpu.make_async_copy(
                tab.at[pl.ds(0, half)], stage, sem_s).wait()
            pltpu.sync_copy(stage, out.at[pl.ds(base, half)])
            pltpu.make_async_copy(
                tab.at[pl.ds(0, half)],
                out.at[pl.ds(base + half, half)], sem_d).wait()

    return gather


def _dense_body(cell_ref, gene_ref, w_ref, qz_ref, pz_ref, recon_ref):
    c = cell_ref[...]
    g = gene_ref[...]
    W = w_ref[...]
    dn = (((1,), (1,)), ((), ()))
    pzl = lax.dot_general(c, W, dn, preferred_element_type=jnp.float32)
    qzl = lax.dot_general(c * g, W, dn, preferred_element_type=jnp.float32)
    mq = jnp.max(qzl, axis=-1, keepdims=True)
    mp = jnp.max(pzl, axis=-1, keepdims=True)
    eq = jnp.exp(qzl - mq)
    ep = jnp.exp(pzl - mp)
    qz_ref[...] = eq / jnp.sum(eq, axis=-1, keepdims=True)
    pz_ref[...] = ep / jnp.sum(ep, axis=-1, keepdims=True)
    iota = lax.broadcasted_iota(jnp.int32, qzl.shape, 1)
    cand = jnp.where(qzl == mq, iota, LABELS)
    am = jnp.min(cand, axis=-1, keepdims=True)
    onehot = (iota == am).astype(jnp.float32)
    recon_ref[...] = lax.dot_general(
        onehot, W, (((1,), (0,)), ((), ())), preferred_element_type=jnp.float32)


def kernel(cells, genes, w_cell_table, w_gene_table, W_ct):
    B = cells.shape[0]
    cell_emb, gene_emb = _make_gather(B)(
        cells, genes, w_cell_table, w_gene_table)

    blk = 2048
    qz, pz, recon = pl.pallas_call(
        _dense_body,
        grid=(B // blk,),
        in_specs=[
            pl.BlockSpec((blk, EMB_DIM), lambda i: (i, 0)),
            pl.BlockSpec((blk, EMB_DIM), lambda i: (i, 0)),
            pl.BlockSpec((LABELS, EMB_DIM), lambda i: (0, 0)),
        ],
        out_specs=[
            pl.BlockSpec((blk, LABELS), lambda i: (i, 0)),
            pl.BlockSpec((blk, LABELS), lambda i: (i, 0)),
            pl.BlockSpec((blk, EMB_DIM), lambda i: (i, 0)),
        ],
        out_shape=[
            jax.ShapeDtypeStruct((B, LABELS), jnp.float32),
            jax.ShapeDtypeStruct((B, LABELS), jnp.float32),
            jax.ShapeDtypeStruct((B, EMB_DIM), jnp.float32),
        ],
    )(cell_emb, gene_emb, W_ct)
    return (qz, pz, cell_emb, recon)


# speedup vs baseline: 1.5398x; 1.5398x over previous
import functools

import jax
import jax.numpy as jnp
from jax import lax
from jax.experimental import pallas as pl
from jax.experimental.pallas import tpu as pltpu
from jax.experimental.pallas import tpu_sc as plsc

EMB_DIM = 64
LABELS = 64

_NC = 2
_NS = 16
_NW = _NC * _NS


@functools.lru_cache(maxsize=None)
def _make_gather(B):
    b_per_w = B // _NW
    mesh = plsc.VectorSubcoreMesh(core_axis_name="c", subcore_axis_name="s")

    @functools.partial(
        pl.kernel,
        mesh=mesh,
        out_type=[
            jax.ShapeDtypeStruct((B, EMB_DIM), jnp.float32),
            jax.ShapeDtypeStruct((B, EMB_DIM), jnp.float32),
        ],
        scratch_types=[
            pltpu.VMEM((b_per_w,), jnp.int32),
            pltpu.VMEM((b_per_w,), jnp.int32),
            pltpu.VMEM((b_per_w, EMB_DIM), jnp.float32),
            pltpu.SemaphoreType.DMA,
        ],
    )
    def gather(cells_hbm, genes_hbm, cell_tab, gene_tab, cell_out, gene_out,
               idx_c, idx_g, stage, sem):
        wid = lax.axis_index("s") * _NC + lax.axis_index("c")
        base = wid * b_per_w
        pltpu.sync_copy(cells_hbm.at[pl.ds(base, b_per_w)], idx_c)
        pltpu.sync_copy(genes_hbm.at[pl.ds(base, b_per_w)], idx_g)

        for tab, idx, out in (
            (cell_tab, idx_c, cell_out),
            (gene_tab, idx_g, gene_out),
        ):
            def body(g, carry):
                j0 = g * 16
                vec = idx[pl.ds(j0, 16)]
                for k in range(16):
                    pltpu.make_async_copy(
                        tab.at[pl.ds(vec[k], 1)],
                        stage.at[pl.ds(j0 + k, 1)], sem).start()
                return carry

            lax.fori_loop(0, b_per_w // 16, body, 0)
            pltpu.make_async_copy(
                tab.at[pl.ds(0, b_per_w)], stage, sem).wait()
            pltpu.sync_copy(stage, out.at[pl.ds(base, b_per_w)])

    return gather


def _dense_body(cell_ref, gene_ref, w_ref, qz_ref, pz_ref, recon_ref):
    c = cell_ref[...]
    g = gene_ref[...]
    W = w_ref[...]
    dn = (((1,), (1,)), ((), ()))
    pzl = lax.dot_general(c, W, dn, preferred_element_type=jnp.float32)
    qzl = lax.dot_general(c * g, W, dn, preferred_element_type=jnp.float32)
    mq = jnp.max(qzl, axis=-1, keepdims=True)
    mp = jnp.max(pzl, axis=-1, keepdims=True)
    eq = jnp.exp(qzl - mq)
    ep = jnp.exp(pzl - mp)
    qz_ref[...] = eq / jnp.sum(eq, axis=-1, keepdims=True)
    pz_ref[...] = ep / jnp.sum(ep, axis=-1, keepdims=True)
    iota = lax.broadcasted_iota(jnp.int32, qzl.shape, 1)
    cand = jnp.where(qzl == mq, iota, LABELS)
    am = jnp.min(cand, axis=-1, keepdims=True)
    onehot = (iota == am).astype(jnp.float32)
    recon_ref[...] = lax.dot_general(
        onehot, W, (((1,), (0,)), ((), ())), preferred_element_type=jnp.float32)


def kernel(cells, genes, w_cell_table, w_gene_table, W_ct):
    B = cells.shape[0]
    cell_emb, gene_emb = _make_gather(B)(
        cells, genes, w_cell_table, w_gene_table)

    blk = 2048
    qz, pz, recon = pl.pallas_call(
        _dense_body,
        grid=(B // blk,),
        in_specs=[
            pl.BlockSpec((blk, EMB_DIM), lambda i: (i, 0)),
            pl.BlockSpec((blk, EMB_DIM), lambda i: (i, 0)),
            pl.BlockSpec((LABELS, EMB_DIM), lambda i: (0, 0)),
        ],
        out_specs=[
            pl.BlockSpec((blk, LABELS), lambda i: (i, 0)),
            pl.BlockSpec((blk, LABELS), lambda i: (i, 0)),
            pl.BlockSpec((blk, EMB_DIM), lambda i: (i, 0)),
        ],
        out_shape=[
            jax.ShapeDtypeStruct((B, LABELS), jnp.float32),
            jax.ShapeDtypeStruct((B, LABELS), jnp.float32),
            jax.ShapeDtypeStruct((B, EMB_DIM), jnp.float32),
        ],
    )(cell_emb, gene_emb, W_ct)
    return (qz, pz, cell_emb, recon)
